# baseline (speedup 1.0000x reference)
import functools

import jax
import jax.numpy as jnp
from jax import lax
from jax.experimental import pallas as pl
from jax.experimental.pallas import tpu as pltpu
from jax.experimental.pallas import tpu_sc as plsc

_LANES = 16


def _index_chunks(length):
  chunks = []
  off = 0
  while off < length:
    size = min(128, length - off)
    chunks.append((off, size))
    off += size
  return chunks


@functools.lru_cache(maxsize=None)
def _build(B, L, V, D, C):
  assert D % _LANES == 0
  KD = D // _LANES
  NC, NS = 2, 16
  NW = NC * NS
  assert B % NW == 0
  BPW = B // NW
  inv_l = 1.0 / L
  chunks = _index_chunks(L)
  UNROLL = 8
  assert L % UNROLL == 0

  mesh = plsc.VectorSubcoreMesh(core_axis_name="c", subcore_axis_name="s")

  @functools.partial(
      pl.kernel,
      out_type=jax.ShapeDtypeStruct((B, _LANES), jnp.float32),
      mesh=mesh,
      compiler_params=pltpu.CompilerParams(use_tc_tiling_on_sc=False),
      scratch_types=[
          pltpu.VMEM((len(chunks), 128), jnp.int32),
          pltpu.VMEM((L, D), jnp.float32),
          pltpu.VMEM((D, _LANES), jnp.float32),
          pltpu.VMEM((_LANES,), jnp.float32),
          pltpu.VMEM((BPW, _LANES), jnp.float32),
          pltpu.SemaphoreType.DMA,
      ],
  )
  def sc_kernel(text_hbm, table_hbm, fcwt_hbm, fcb_hbm, out_hbm,
                idx_v, rows_v, fcwt_v, fcb_v, out_v, sem):
    wid = lax.axis_index("s") * NC + lax.axis_index("c")
    base = wid * BPW
    pltpu.sync_copy(fcwt_hbm, fcwt_v)
    pltpu.sync_copy(fcb_hbm, fcb_v)
    fcb_vec = fcb_v[pl.ds(0, _LANES)]

    def row_body(r, carry):
      row = base + r
      for j, (off, size) in enumerate(chunks):
        pltpu.sync_copy(text_hbm.at[row, pl.ds(off, size)],
                        idx_v.at[j, pl.ds(0, size)])
      copies = []
      for j, (off, size) in enumerate(chunks):
        copies.append(
            pltpu.async_copy(table_hbm.at[idx_v.at[j, pl.ds(0, size)]],
                             rows_v.at[pl.ds(off, size)], sem))
      for cp in copies:
        cp.wait()

      zero = jnp.zeros((_LANES,), jnp.float32)

      def red_body(i, accs):
        ib = i * UNROLL
        accs = list(accs)
        for u in range(UNROLL):
          for k in range(KD):
            accs[k] = accs[k] + rows_v[ib + u, pl.ds(k * _LANES, _LANES)]
        return tuple(accs)

      accs = lax.fori_loop(0, L // UNROLL, red_body, (zero,) * KD)
      pooled = [a * inv_l for a in accs]

      parts = [fcb_vec, zero, zero, zero]
      for d in range(D):
        parts[d % 4] = parts[d % 4] + (
            pooled[d // _LANES][d % _LANES] * fcwt_v[d, pl.ds(0, _LANES)])
      out_row = (parts[0] + parts[1]) + (parts[2] + parts[3])
      out_v[r, pl.ds(0, _LANES)] = out_row
      return carry

    lax.fori_loop(0, BPW, row_body, 0)
    pltpu.sync_copy(out_v, out_hbm.at[pl.ds(base, BPW)])

  return sc_kernel


def kernel(text, emb_table, fc_w, fc_b):
  B, L = text.shape
  V, D = emb_table.shape
  C = fc_w.shape[0]
  text = text.astype(jnp.int32)
  fcwt = jnp.zeros((D, _LANES), jnp.float32).at[:, :C].set(fc_w.T)
  fcb_pad = jnp.zeros((_LANES,), jnp.float32).at[:C].set(fc_b)
  out = _build(B, L, V, D, C)(text, emb_table, fcwt, fcb_pad)
  return out[:, :C]


# speedup vs baseline: 1.2808x; 1.2808x over previous
import functools

import jax
import jax.numpy as jnp
from jax import lax
from jax.experimental import pallas as pl
from jax.experimental.pallas import tpu as pltpu
from jax.experimental.pallas import tpu_sc as plsc

_LANES = 16
_IDXW = 100


@functools.lru_cache(maxsize=None)
def _build(B, L, V, D, C):
  assert D % _LANES == 0
  KD = D // _LANES
  NC, NS = 2, 16
  NW = NC * NS
  assert B % NW == 0
  BPW = B // NW
  assert L % (2 * _IDXW) == 0 or L == 2 * _IDXW
  SPLITS = L // _IDXW
  inv_l = 1.0 / L
  UNROLL = 25
  assert L % UNROLL == 0

  mesh = plsc.VectorSubcoreMesh(core_axis_name="c", subcore_axis_name="s")

  @functools.partial(
      pl.kernel,
      out_type=jax.ShapeDtypeStruct((B, _LANES), jnp.float32),
      mesh=mesh,
      compiler_params=pltpu.CompilerParams(use_tc_tiling_on_sc=False),
      scratch_types=[
          pltpu.VMEM((BPW * SPLITS, _IDXW), jnp.int32),
          pltpu.VMEM((L, D), jnp.float32),
          pltpu.VMEM((L, D), jnp.float32),
          pltpu.VMEM((D, _LANES), jnp.float32),
          pltpu.VMEM((_LANES,), jnp.float32),
          pltpu.VMEM((BPW, _LANES), jnp.float32),
          pltpu.SemaphoreType.DMA,
          pltpu.SemaphoreType.DMA,
      ],
  )
  def sc_kernel(text_hbm, table_hbm, fcwt_hbm, fcb_hbm, out_hbm,
                idx_v, rows0_v, rows1_v, fcwt_v, fcb_v, out_v, sem0, sem1):
    wid = lax.axis_index("s") * NC + lax.axis_index("c")
    base = wid * BPW
    bufs = (rows0_v, rows1_v)
    sems = (sem0, sem1)

    pltpu.sync_copy(fcwt_hbm, fcwt_v)
    pltpu.sync_copy(fcb_hbm, fcb_v)
    pltpu.sync_copy(text_hbm.at[pl.ds(base * SPLITS, BPW * SPLITS)], idx_v)
    fcb_vec = fcb_v[pl.ds(0, _LANES)]

    def fire(r, buf, sem):
      for j in range(SPLITS):
        pltpu.async_copy(
            table_hbm.at[idx_v.at[r * SPLITS + j]],
            buf.at[pl.ds(j * _IDXW, _IDXW)], sem)

    def drain(buf, sem):
      pltpu.make_async_copy(table_hbm.at[pl.ds(0, L)], buf, sem).wait()

    def process(r, buf):
      zero = jnp.zeros((_LANES,), jnp.float32)

      def red_body(i, accs):
        ib = i * UNROLL
        accs = list(accs)
        for u in range(UNROLL):
          for k in range(KD):
            accs[k] = accs[k] + buf[ib + u, pl.ds(k * _LANES, _LANES)]
        return tuple(accs)

      accs = lax.fori_loop(0, L // UNROLL, red_body, (zero,) * KD)
      pooled = [a * inv_l for a in accs]

      parts = [fcb_vec, zero, zero, zero]
      for d in range(D):
        parts[d % 4] = parts[d % 4] + (
            pooled[d // _LANES][d % _LANES] * fcwt_v[d, pl.ds(0, _LANES)])
      out_row = (parts[0] + parts[1]) + (parts[2] + parts[3])
      out_v[r, pl.ds(0, _LANES)] = out_row

    fire(0, bufs[0], sems[0])

    def body(g, carry):
      r0 = 2 * g
      r1 = 2 * g + 1
      fire(r1, bufs[1], sems[1])
      drain(bufs[0], sems[0])
      process(r0, bufs[0])

      @pl.when(r1 + 1 < BPW)
      def _():
        fire(r1 + 1, bufs[0], sems[0])

      drain(bufs[1], sems[1])
      process(r1, bufs[1])
      return carry

    lax.fori_loop(0, BPW // 2, body, 0)
    pltpu.sync_copy(out_v, out_hbm.at[pl.ds(base, BPW)])

  return sc_kernel


def kernel(text, emb_table, fc_w, fc_b):
  B, L = text.shape
  V, D = emb_table.shape
  C = fc_w.shape[0]
  text = text.astype(jnp.int32).reshape(B * (L // _IDXW), _IDXW)
  fcwt = jnp.zeros((D, _LANES), jnp.float32).at[:, :C].set(fc_w.T)
  fcb_pad = jnp.zeros((_LANES,), jnp.float32).at[:C].set(fc_b)
  out = _build(B, L, V, D, C)(text, emb_table, fcwt, fcb_pad)
  return out[:, :C]


# speedup vs baseline: 1.3566x; 1.0592x over previous
import functools

import jax
import jax.numpy as jnp
from jax import lax
from jax.experimental import pallas as pl
from jax.experimental.pallas import tpu as pltpu
from jax.experimental.pallas import tpu_sc as plsc

_LANES = 16
_IDXW = 100


@functools.lru_cache(maxsize=None)
def _build(B, L, V, D, C):
  assert D % _LANES == 0
  KD = D // _LANES
  NC, NS = 2, 16
  NW = NC * NS
  assert B % NW == 0
  BPW = B // NW
  assert L % (2 * _IDXW) == 0 or L == 2 * _IDXW
  SPLITS = L // _IDXW
  inv_l = 1.0 / L
  UNROLL = 25
  assert L % UNROLL == 0

  mesh = plsc.VectorSubcoreMesh(core_axis_name="c", subcore_axis_name="s")

  @functools.partial(
      pl.kernel,
      out_type=jax.ShapeDtypeStruct((B, _LANES), jnp.float32),
      mesh=mesh,
      compiler_params=pltpu.CompilerParams(use_tc_tiling_on_sc=False),
      scratch_types=[
          pltpu.VMEM((BPW * SPLITS, _IDXW), jnp.int32),
          pltpu.VMEM((L, D), jnp.float32),
          pltpu.VMEM((L, D), jnp.float32),
          pltpu.VMEM((L, D), jnp.float32),
          pltpu.VMEM((L, D), jnp.float32),
          pltpu.VMEM((D, _LANES), jnp.float32),
          pltpu.VMEM((_LANES,), jnp.float32),
          pltpu.VMEM((BPW, _LANES), jnp.float32),
          pltpu.SemaphoreType.DMA,
          pltpu.SemaphoreType.DMA,
          pltpu.SemaphoreType.DMA,
          pltpu.SemaphoreType.DMA,
      ],
  )
  def sc_kernel(text_hbm, table_hbm, fcwt_hbm, fcb_hbm, out_hbm,
                idx_v, rows0_v, rows1_v, rows2_v, rows3_v,
                fcwt_v, fcb_v, out_v, sem0, sem1, sem2, sem3):
    wid = lax.axis_index("s") * NC + lax.axis_index("c")
    base = wid * BPW
    bufs = (rows0_v, rows1_v, rows2_v, rows3_v)
    sems = (sem0, sem1, sem2, sem3)
    NBUF = 4
    assert BPW % NBUF == 0

    pltpu.sync_copy(fcwt_hbm, fcwt_v)
    pltpu.sync_copy(fcb_hbm, fcb_v)
    pltpu.sync_copy(text_hbm.at[pl.ds(base * SPLITS, BPW * SPLITS)], idx_v)
    fcb_vec = fcb_v[pl.ds(0, _LANES)]

    def fire(r, buf, sem):
      for j in range(SPLITS):
        pltpu.async_copy(
            table_hbm.at[idx_v.at[r * SPLITS + j]],
            buf.at[pl.ds(j * _IDXW, _IDXW)], sem)

    def drain(buf, sem):
      pltpu.make_async_copy(table_hbm.at[pl.ds(0, L)], buf, sem).wait()

    def process(r, buf):
      zero = jnp.zeros((_LANES,), jnp.float32)

      def red_body(i, accs):
        ib = i * UNROLL
        accs = list(accs)
        for u in range(UNROLL):
          for k in range(KD):
            accs[k] = accs[k] + buf[ib + u, pl.ds(k * _LANES, _LANES)]
        return tuple(accs)

      accs = lax.fori_loop(0, L // UNROLL, red_body, (zero,) * KD)
      pooled = [a * inv_l for a in accs]

      parts = [fcb_vec, zero, zero, zero]
      for d in range(D):
        parts[d % 4] = parts[d % 4] + (
            pooled[d // _LANES][d % _LANES] * fcwt_v[d, pl.ds(0, _LANES)])
      out_row = (parts[0] + parts[1]) + (parts[2] + parts[3])
      out_v[r, pl.ds(0, _LANES)] = out_row

    for b in range(NBUF - 1):
      fire(b, bufs[b], sems[b])

    def body(g, carry):
      for b in range(NBUF):
        r = g * NBUF + b
        nxt = r + NBUF - 1
        pb = (b - 1) % NBUF

        @pl.when(nxt < BPW)
        def _():
          fire(nxt, bufs[pb], sems[pb])

        drain(bufs[b], sems[b])
        process(r, bufs[b])
      return carry

    lax.fori_loop(0, BPW // NBUF, body, 0)
    pltpu.sync_copy(out_v, out_hbm.at[pl.ds(base, BPW)])

  return sc_kernel


def kernel(text, emb_table, fc_w, fc_b):
  B, L = text.shape
  V, D = emb_table.shape
  C = fc_w.shape[0]
  text = text.astype(jnp.int32).reshape(B * (L // _IDXW), _IDXW)
  fcwt = jnp.zeros((D, _LANES), jnp.float32).at[:, :C].set(fc_w.T)
  fcb_pad = jnp.zeros((_LANES,), jnp.float32).at[:C].set(fc_b)
  out = _build(B, L, V, D, C)(text, emb_table, fcwt, fcb_pad)
  return out[:, :C]
